# baseline (speedup 1.0000x reference)
import functools

import jax
import jax.numpy as jnp
from jax import lax
from jax.experimental import pallas as pl
from jax.experimental.pallas import tpu as pltpu
from jax.experimental.pallas import tpu_sc as plsc

N = 10000
E = 320000
D = 128

NC = 2
NS = 16
NW = NC * NS
EPW = E // NW
K = 128
CPW = 80
CPP = 16
EPAD = NW * CPW * K
NPAD = 10240
RPW = NPAD // NS
NBINS = 16384
BPT = NBINS // NS

_mesh = plsc.VectorSubcoreMesh(
    core_axis_name="c", subcore_axis_name="s", num_cores=NC, num_subcores=NS
)


@functools.partial(
    pl.kernel,
    out_type=jax.ShapeDtypeStruct((NC, NBINS), jnp.float32),
    mesh=_mesh,
    scratch_types=[
        pltpu.VMEM((NBINS,), jnp.int32),
        pltpu.VMEM((EPW,), jnp.int32),
        pltpu.VMEM((NS, BPT), jnp.int32),
        pltpu.VMEM((BPT,), jnp.float32),
        pltpu.VMEM_SHARED((NS, NBINS), jnp.int32),
    ],
    compiler_params=pltpu.CompilerParams(needs_layout_passes=False),
)
def _deg_call(dst_hbm, out_hbm, hist_v, idx_v, red_v, res_v, stage_sh):
    cid = lax.axis_index("c")
    sid = lax.axis_index("s")
    wid = sid * NC + cid

    zero16 = jnp.zeros((16,), jnp.int32)
    ones16 = jnp.ones((16,), jnp.int32)

    pltpu.sync_copy(dst_hbm.at[pl.ds(pl.multiple_of(wid * EPW, EPW), EPW)], idx_v)

    def _zb(i, carry):
        hist_v[pl.ds(pl.multiple_of(i * 16, 16), 16)] = zero16
        return carry

    lax.fori_loop(0, NBINS // 16, _zb, 0)

    def _acc(i, carry):
        idx16 = idx_v[pl.ds(pl.multiple_of(i * 16, 16), 16)]
        plsc.addupdate_scatter(hist_v, [idx16], ones16)
        return carry

    lax.fori_loop(0, EPW // 16, _acc, 0)

    pltpu.sync_copy(hist_v, stage_sh.at[sid])
    plsc.subcore_barrier()
    pltpu.sync_copy(stage_sh.at[:, pl.ds(sid * BPT, BPT)], red_v)

    def _red(i, carry):
        off = pl.ds(pl.multiple_of(i * 16, 16), 16)
        acc = zero16

        def _rows(r, a):
            return a + red_v[r, off]

        acc = lax.fori_loop(0, NS, _rows, acc)
        res_v[off] = acc.astype(jnp.float32)
        return carry

    lax.fori_loop(0, BPT // 16, _red, 0)
    pltpu.sync_copy(res_v, out_hbm.at[cid, pl.ds(sid * BPT, BPT)])


NBUF = 2


@functools.partial(
    pl.kernel,
    out_type=jax.ShapeDtypeStruct((NC, NPAD, D), jnp.float32),
    mesh=_mesh,
    scratch_types=[
        pltpu.VMEM((CPP, K), jnp.int32),
        pltpu.VMEM((CPP, K), jnp.int32),
        pltpu.VMEM((NBUF, K, D), jnp.float32),
        pltpu.VMEM_SHARED((NPAD, D), jnp.float32),
        pltpu.SemaphoreType.DMA((NBUF,)),
        pltpu.SemaphoreType.DMA((NBUF,)),
    ],
    compiler_params=pltpu.CompilerParams(needs_layout_passes=False),
)
def _scatter_call(g_hbm, src_hbm, dst_hbm, out_hbm, srcv, dstv, ring, acc, gsem, ssem):
    cid = lax.axis_index("c")
    sid = lax.axis_index("s")
    wid = sid * NC + cid

    zero16 = jnp.zeros((16,), jnp.float32)

    def _zb(i, carry):
        ring[0, i >> 3, pl.ds(pl.multiple_of((i & 7) * 16, 16), 16)] = zero16
        return carry

    lax.fori_loop(0, K * (D // 16), _zb, 0)

    def _za(k, carry):
        pltpu.sync_copy(ring.at[0], acc.at[pl.ds(sid * RPW + k * K, K)])
        return carry

    lax.fori_loop(0, RPW // K, _za, 0)
    plsc.subcore_barrier()

    cbase = wid * CPW

    def _gather(c, b):
        pltpu.async_copy(g_hbm.at[srcv.at[c]], ring.at[b], gsem.at[b])

    def _gwait(c, b):
        pltpu.make_async_copy(g_hbm.at[srcv.at[c]], ring.at[b], gsem.at[b]).wait()

    def _scat(c, b):
        pltpu.async_copy(ring.at[b], acc.at[dstv.at[c]], ssem.at[b], add=True)

    def _swait(c, b):
        pltpu.make_async_copy(ring.at[b], acc.at[dstv.at[c]], ssem.at[b]).wait()

    def _phase(p, carry):
        prow = pl.multiple_of(cbase + p * CPP, 8)
        pltpu.sync_copy(src_hbm.at[pl.ds(prow, CPP)], srcv)
        pltpu.sync_copy(dst_hbm.at[pl.ds(prow, CPP)], dstv)

        def _chunk(c, c2):
            pltpu.sync_copy(g_hbm.at[srcv.at[c]], ring.at[0])
            pltpu.sync_copy(ring.at[0], acc.at[dstv.at[c]], add=True)
            return c2

        lax.fori_loop(0, CPP, _chunk, 0)
        return carry

    lax.fori_loop(0, CPW // CPP, _phase, 0)
    plsc.subcore_barrier()

    pltpu.sync_copy(
        acc.at[pl.ds(sid * RPW, RPW)], out_hbm.at[cid, pl.ds(sid * RPW, RPW)]
    )


_ROWS_BLK = 2000


def _mm_body(x_ref, w_ref, deg_ref, g_ref):
    dinv = lax.rsqrt(deg_ref[...])
    g_ref[...] = (
        jnp.dot(x_ref[...], w_ref[...], preferred_element_type=jnp.float32)
        * dinv
    )


def _fin_body(s0_ref, s1_ref, g_ref, deg_ref, b_ref, o_ref):
    dinv = lax.rsqrt(deg_ref[...])
    z = (s0_ref[...] + s1_ref[...] + g_ref[...]) * dinv + b_ref[...]
    o_ref[...] = jnp.where(z > 0, z, jnp.exp(jnp.minimum(z, 0.0)) - 1.0)


def kernel(x, edge_index, W, b):
    src = edge_index[0]
    dst = edge_index[1]
    pad = EPAD - E
    src2d = jnp.concatenate([src, jnp.zeros((pad,), src.dtype)]).reshape(
        EPAD // K, K
    )
    pad_dst = N + (jnp.arange(pad, dtype=dst.dtype) % (NPAD - N))
    dst2d = jnp.concatenate([dst, pad_dst]).reshape(EPAD // K, K)

    hp = _deg_call(dst)
    deg_col = ((hp[0] + hp[1])[:N] + 1.0).reshape(N, 1)

    g = pl.pallas_call(
        _mm_body,
        grid=(N // _ROWS_BLK,),
        in_specs=[
            pl.BlockSpec((_ROWS_BLK, D), lambda i: (i, 0)),
            pl.BlockSpec((D, D), lambda i: (0, 0)),
            pl.BlockSpec((_ROWS_BLK, 1), lambda i: (i, 0)),
        ],
        out_specs=pl.BlockSpec((_ROWS_BLK, D), lambda i: (i, 0)),
        out_shape=jax.ShapeDtypeStruct((N, D), jnp.float32),
    )(x, W, deg_col)

    s = _scatter_call(g, src2d, dst2d)
    s0 = s[0, :N]
    s1 = s[1, :N]

    out = pl.pallas_call(
        _fin_body,
        grid=(N // _ROWS_BLK,),
        in_specs=[
            pl.BlockSpec((_ROWS_BLK, D), lambda i: (i, 0)),
            pl.BlockSpec((_ROWS_BLK, D), lambda i: (i, 0)),
            pl.BlockSpec((_ROWS_BLK, D), lambda i: (i, 0)),
            pl.BlockSpec((_ROWS_BLK, 1), lambda i: (i, 0)),
            pl.BlockSpec((1, D), lambda i: (0, 0)),
        ],
        out_specs=pl.BlockSpec((_ROWS_BLK, D), lambda i: (i, 0)),
        out_shape=jax.ShapeDtypeStruct((N, D), jnp.float32),
    )(s0, s1, g, deg_col, b.reshape(1, D))

    return out


# speedup vs baseline: 1.1340x; 1.1340x over previous
import functools

import jax
import jax.numpy as jnp
from jax import lax
from jax.experimental import pallas as pl
from jax.experimental.pallas import tpu as pltpu
from jax.experimental.pallas import tpu_sc as plsc

N = 10000
E = 320000
D = 128

NC = 2
NS = 16
NW = NC * NS
EPW = E // NW
K = 128
CPW = 80
CPP = 16
EPAD = NW * CPW * K
NPAD = 10240
RPW = NPAD // NS
NBINS = 16384
BPT = NBINS // NS

_mesh = plsc.VectorSubcoreMesh(
    core_axis_name="c", subcore_axis_name="s", num_cores=NC, num_subcores=NS
)


@functools.partial(
    pl.kernel,
    out_type=jax.ShapeDtypeStruct((NC, NBINS), jnp.float32),
    mesh=_mesh,
    scratch_types=[
        pltpu.VMEM((NBINS,), jnp.int32),
        pltpu.VMEM((EPW,), jnp.int32),
        pltpu.VMEM((NS, BPT), jnp.int32),
        pltpu.VMEM((BPT,), jnp.float32),
        pltpu.VMEM_SHARED((NS, NBINS), jnp.int32),
    ],
    compiler_params=pltpu.CompilerParams(needs_layout_passes=False),
)
def _deg_call(dst_hbm, out_hbm, hist_v, idx_v, red_v, res_v, stage_sh):
    cid = lax.axis_index("c")
    sid = lax.axis_index("s")
    wid = sid * NC + cid

    zero16 = jnp.zeros((16,), jnp.int32)
    ones16 = jnp.ones((16,), jnp.int32)

    pltpu.sync_copy(dst_hbm.at[pl.ds(pl.multiple_of(wid * EPW, EPW), EPW)], idx_v)

    def _zb(i, carry):
        hist_v[pl.ds(pl.multiple_of(i * 16, 16), 16)] = zero16
        return carry

    lax.fori_loop(0, NBINS // 16, _zb, 0)

    def _acc(i, carry):
        idx16 = idx_v[pl.ds(pl.multiple_of(i * 16, 16), 16)]
        plsc.addupdate_scatter(hist_v, [idx16], ones16)
        return carry

    lax.fori_loop(0, EPW // 16, _acc, 0)

    pltpu.sync_copy(hist_v, stage_sh.at[sid])
    plsc.subcore_barrier()
    pltpu.sync_copy(stage_sh.at[:, pl.ds(sid * BPT, BPT)], red_v)

    def _red(i, carry):
        off = pl.ds(pl.multiple_of(i * 16, 16), 16)
        acc = zero16

        def _rows(r, a):
            return a + red_v[r, off]

        acc = lax.fori_loop(0, NS, _rows, acc)
        res_v[off] = acc.astype(jnp.float32)
        return carry

    lax.fori_loop(0, BPT // 16, _red, 0)
    pltpu.sync_copy(res_v, out_hbm.at[cid, pl.ds(sid * BPT, BPT)])


NBUF = 2


@functools.partial(
    pl.kernel,
    out_type=jax.ShapeDtypeStruct((NC, NPAD, D), jnp.float32),
    mesh=_mesh,
    scratch_types=[
        pltpu.VMEM((CPP, K), jnp.int32),
        pltpu.VMEM((CPP, K), jnp.int32),
        pltpu.VMEM((NBUF, K, D), jnp.float32),
        pltpu.VMEM_SHARED((NPAD, D), jnp.float32),
        pltpu.SemaphoreType.DMA((NBUF,)),
        pltpu.SemaphoreType.DMA((NBUF,)),
    ],
    compiler_params=pltpu.CompilerParams(needs_layout_passes=False),
)
def _scatter_call(g_hbm, src_hbm, dst_hbm, out_hbm, srcv, dstv, ring, acc, gsem, ssem):
    cid = lax.axis_index("c")
    sid = lax.axis_index("s")
    wid = sid * NC + cid

    zero16 = jnp.zeros((16,), jnp.float32)

    def _zb(i, carry):
        ring[0, i >> 3, pl.ds(pl.multiple_of((i & 7) * 16, 16), 16)] = zero16
        return carry

    lax.fori_loop(0, K * (D // 16), _zb, 0)

    def _za(k, carry):
        pltpu.sync_copy(ring.at[0], acc.at[pl.ds(sid * RPW + k * K, K)])
        return carry

    lax.fori_loop(0, RPW // K, _za, 0)
    plsc.subcore_barrier()

    cbase = wid * CPW

    def _gather(c, b):
        pltpu.async_copy(g_hbm.at[srcv.at[c]], ring.at[b], gsem.at[b])

    def _gwait(c, b):
        pltpu.make_async_copy(g_hbm.at[srcv.at[c]], ring.at[b], gsem.at[b]).wait()

    def _scat(c, b):
        pltpu.async_copy(ring.at[b], acc.at[dstv.at[c]], ssem.at[b], add=True)

    def _swait(c, b):
        pltpu.make_async_copy(ring.at[b], acc.at[dstv.at[c]], ssem.at[b]).wait()

    def _phase(p, carry):
        prow = pl.multiple_of(cbase + p * CPP, 8)
        pltpu.sync_copy(src_hbm.at[pl.ds(prow, CPP)], srcv)
        pltpu.sync_copy(dst_hbm.at[pl.ds(prow, CPP)], dstv)

        def _chunk(c, c2):
            pltpu.sync_copy(g_hbm.at[srcv.at[c]], ring.at[0])
            pltpu.sync_copy(ring.at[0], acc.at[dstv.at[c]], add=True)
            return c2

        lax.fori_loop(0, CPP, _chunk, 0)
        return carry

    lax.fori_loop(0, CPW // CPP, _phase, 0)
    plsc.subcore_barrier()

    pltpu.sync_copy(
        acc.at[pl.ds(sid * RPW, RPW)], out_hbm.at[cid, pl.ds(sid * RPW, RPW)]
    )


_ROWS_BLK = 2000


def _mm_body(x_ref, w_ref, deg_ref, g_ref):
    dinv = lax.rsqrt(deg_ref[...])
    g_ref[...] = (
        jnp.dot(x_ref[...], w_ref[...], preferred_element_type=jnp.float32)
        * dinv
    )


def _fin_body(s0_ref, s1_ref, g_ref, deg_ref, b_ref, o_ref):
    dinv = lax.rsqrt(deg_ref[...])
    z = (s0_ref[...] + s1_ref[...] + g_ref[...]) * dinv + b_ref[...]
    o_ref[...] = jnp.where(z > 0, z, jnp.exp(jnp.minimum(z, 0.0)) - 1.0)


def kernel(x, edge_index, W, b):
    src = edge_index[0]
    dst = edge_index[1]
    pad = EPAD - E
    src2d = jnp.concatenate([src, jnp.zeros((pad,), src.dtype)]).reshape(
        EPAD // K, K
    )
    pad_dst = N + (jnp.arange(pad, dtype=dst.dtype) % (NPAD - N))
    dst2d = jnp.concatenate([dst, pad_dst]).reshape(EPAD // K, K)
    src2d = src2d.reshape(CPW, NW, K).transpose(1, 0, 2).reshape(EPAD // K, K)
    dst2d = dst2d.reshape(CPW, NW, K).transpose(1, 0, 2).reshape(EPAD // K, K)

    hp = _deg_call(dst)
    deg_col = ((hp[0] + hp[1])[:N] + 1.0).reshape(N, 1)

    g = pl.pallas_call(
        _mm_body,
        grid=(N // _ROWS_BLK,),
        in_specs=[
            pl.BlockSpec((_ROWS_BLK, D), lambda i: (i, 0)),
            pl.BlockSpec((D, D), lambda i: (0, 0)),
            pl.BlockSpec((_ROWS_BLK, 1), lambda i: (i, 0)),
        ],
        out_specs=pl.BlockSpec((_ROWS_BLK, D), lambda i: (i, 0)),
        out_shape=jax.ShapeDtypeStruct((N, D), jnp.float32),
    )(x, W, deg_col)

    s = _scatter_call(g, src2d, dst2d)
    s0 = s[0, :N]
    s1 = s[1, :N]

    out = pl.pallas_call(
        _fin_body,
        grid=(N // _ROWS_BLK,),
        in_specs=[
            pl.BlockSpec((_ROWS_BLK, D), lambda i: (i, 0)),
            pl.BlockSpec((_ROWS_BLK, D), lambda i: (i, 0)),
            pl.BlockSpec((_ROWS_BLK, D), lambda i: (i, 0)),
            pl.BlockSpec((_ROWS_BLK, 1), lambda i: (i, 0)),
            pl.BlockSpec((1, D), lambda i: (0, 0)),
        ],
        out_specs=pl.BlockSpec((_ROWS_BLK, D), lambda i: (i, 0)),
        out_shape=jax.ShapeDtypeStruct((N, D), jnp.float32),
    )(s0, s1, g, deg_col, b.reshape(1, D))

    return out


# speedup vs baseline: 1.1943x; 1.0532x over previous
import functools

import jax
import jax.numpy as jnp
from jax import lax
from jax.experimental import pallas as pl
from jax.experimental.pallas import tpu as pltpu
from jax.experimental.pallas import tpu_sc as plsc

N = 10000
E = 320000
D = 128

NC = 2
NS = 16
NW = NC * NS
EPW = E // NW
K = 128
CPW = 80
CPP = 16
EPAD = NW * CPW * K
NPAD = 10240
RPW = NPAD // NS
NBINS = 16384
BPT = NBINS // NS

_mesh = plsc.VectorSubcoreMesh(
    core_axis_name="c", subcore_axis_name="s", num_cores=NC, num_subcores=NS
)


@functools.partial(
    pl.kernel,
    out_type=jax.ShapeDtypeStruct((NC, NBINS), jnp.float32),
    mesh=_mesh,
    scratch_types=[
        pltpu.VMEM((NBINS,), jnp.int32),
        pltpu.VMEM((EPW,), jnp.int32),
        pltpu.VMEM((NS, BPT), jnp.int32),
        pltpu.VMEM((BPT,), jnp.float32),
        pltpu.VMEM_SHARED((NS, NBINS), jnp.int32),
    ],
    compiler_params=pltpu.CompilerParams(needs_layout_passes=False),
)
def _deg_call(dst_hbm, out_hbm, hist_v, idx_v, red_v, res_v, stage_sh):
    cid = lax.axis_index("c")
    sid = lax.axis_index("s")
    wid = sid * NC + cid

    zero16 = jnp.zeros((16,), jnp.int32)
    ones16 = jnp.ones((16,), jnp.int32)

    pltpu.sync_copy(dst_hbm.at[pl.ds(pl.multiple_of(wid * EPW, EPW), EPW)], idx_v)

    def _zb(i, carry):
        hist_v[pl.ds(pl.multiple_of(i * 16, 16), 16)] = zero16
        return carry

    lax.fori_loop(0, NBINS // 16, _zb, 0)

    def _acc(i, carry):
        idx16 = idx_v[pl.ds(pl.multiple_of(i * 16, 16), 16)]
        plsc.addupdate_scatter(hist_v, [idx16], ones16)
        return carry

    lax.fori_loop(0, EPW // 16, _acc, 0)

    pltpu.sync_copy(hist_v, stage_sh.at[sid])
    plsc.subcore_barrier()
    pltpu.sync_copy(stage_sh.at[:, pl.ds(sid * BPT, BPT)], red_v)

    def _red(i, carry):
        off = pl.ds(pl.multiple_of(i * 16, 16), 16)
        acc = zero16

        def _rows(r, a):
            return a + red_v[r, off]

        acc = lax.fori_loop(0, NS, _rows, acc)
        res_v[off] = acc.astype(jnp.float32)
        return carry

    lax.fori_loop(0, BPT // 16, _red, 0)
    pltpu.sync_copy(res_v, out_hbm.at[cid, pl.ds(sid * BPT, BPT)])


NBUF = 2


@functools.partial(
    pl.kernel,
    out_type=jax.ShapeDtypeStruct((NC, NPAD, D), jnp.float32),
    mesh=_mesh,
    scratch_types=[
        pltpu.VMEM((CPP, K), jnp.int32),
        pltpu.VMEM((CPP, K), jnp.int32),
        pltpu.VMEM((NBUF, K, D), jnp.float32),
        pltpu.VMEM_SHARED((NPAD, D), jnp.float32),
        pltpu.SemaphoreType.DMA((NBUF,)),
        pltpu.SemaphoreType.DMA((NBUF,)),
    ],
    compiler_params=pltpu.CompilerParams(needs_layout_passes=False),
)
def _scatter_call(g_hbm, src_hbm, dst_hbm, out_hbm, srcv, dstv, ring, acc, gsem, ssem):
    cid = lax.axis_index("c")
    sid = lax.axis_index("s")
    wid = sid * NC + cid

    zero16 = jnp.zeros((16,), jnp.float32)

    def _zb(i, carry):
        ring[0, i >> 3, pl.ds(pl.multiple_of((i & 7) * 16, 16), 16)] = zero16
        return carry

    lax.fori_loop(0, K * (D // 16), _zb, 0)

    def _za(k, carry):
        pltpu.sync_copy(ring.at[0], acc.at[pl.ds(sid * RPW + k * K, K)])
        return carry

    lax.fori_loop(0, RPW // K, _za, 0)
    plsc.subcore_barrier()

    cbase = wid * CPW

    def _gather(c, b):
        pltpu.async_copy(g_hbm.at[srcv.at[c]], ring.at[b], gsem.at[b])

    def _gwait(c, b):
        pltpu.make_async_copy(g_hbm.at[srcv.at[c]], ring.at[b], gsem.at[b]).wait()

    def _scat(c, b):
        pltpu.async_copy(ring.at[b], acc.at[dstv.at[c]], ssem.at[b], add=True)

    def _swait(c, b):
        pltpu.make_async_copy(ring.at[b], acc.at[dstv.at[c]], ssem.at[b]).wait()

    def _phase(p, carry):
        prow = pl.multiple_of(cbase + p * CPP, 8)
        pltpu.sync_copy(src_hbm.at[pl.ds(prow, CPP)], srcv)
        pltpu.sync_copy(dst_hbm.at[pl.ds(prow, CPP)], dstv)

        for b in range(NBUF):
            _gather(b, b)

        def _round(t, c2):
            c0 = t * NBUF
            for b in range(NBUF):
                _gwait(c0 + b, b)
                _scat(c0 + b, b)
            for b in range(NBUF):
                _swait(c0 + b, b)
                _gather(c0 + NBUF + b, b)
            return c2

        lax.fori_loop(0, CPP // NBUF - 1, _round, 0)

        cl = CPP - NBUF
        for b in range(NBUF):
            _gwait(cl + b, b)
            _scat(cl + b, b)
        for b in range(NBUF):
            _swait(cl + b, b)
        return carry

    lax.fori_loop(0, CPW // CPP, _phase, 0)
    plsc.subcore_barrier()

    pltpu.sync_copy(
        acc.at[pl.ds(sid * RPW, RPW)], out_hbm.at[cid, pl.ds(sid * RPW, RPW)]
    )


_ROWS_BLK = 2000


def _mm_body(x_ref, w_ref, deg_ref, g_ref):
    dinv = lax.rsqrt(deg_ref[...])
    g_ref[...] = (
        jnp.dot(x_ref[...], w_ref[...], preferred_element_type=jnp.float32)
        * dinv
    )


def _fin_body(s0_ref, s1_ref, g_ref, deg_ref, b_ref, o_ref):
    dinv = lax.rsqrt(deg_ref[...])
    z = (s0_ref[...] + s1_ref[...] + g_ref[...]) * dinv + b_ref[...]
    o_ref[...] = jnp.where(z > 0, z, jnp.exp(jnp.minimum(z, 0.0)) - 1.0)


def kernel(x, edge_index, W, b):
    src = edge_index[0]
    dst = edge_index[1]
    pad = EPAD - E
    src2d = jnp.concatenate([src, jnp.zeros((pad,), src.dtype)]).reshape(
        EPAD // K, K
    )
    pad_dst = N + (jnp.arange(pad, dtype=dst.dtype) % (NPAD - N))
    dst2d = jnp.concatenate([dst, pad_dst]).reshape(EPAD // K, K)
    src2d = src2d.reshape(CPW, NW, K).transpose(1, 0, 2).reshape(EPAD // K, K)
    dst2d = dst2d.reshape(CPW, NW, K).transpose(1, 0, 2).reshape(EPAD // K, K)

    hp = _deg_call(dst)
    deg_col = ((hp[0] + hp[1])[:N] + 1.0).reshape(N, 1)

    g = pl.pallas_call(
        _mm_body,
        grid=(N // _ROWS_BLK,),
        in_specs=[
            pl.BlockSpec((_ROWS_BLK, D), lambda i: (i, 0)),
            pl.BlockSpec((D, D), lambda i: (0, 0)),
            pl.BlockSpec((_ROWS_BLK, 1), lambda i: (i, 0)),
        ],
        out_specs=pl.BlockSpec((_ROWS_BLK, D), lambda i: (i, 0)),
        out_shape=jax.ShapeDtypeStruct((N, D), jnp.float32),
    )(x, W, deg_col)

    s = _scatter_call(g, src2d, dst2d)
    s0 = s[0, :N]
    s1 = s[1, :N]

    out = pl.pallas_call(
        _fin_body,
        grid=(N // _ROWS_BLK,),
        in_specs=[
            pl.BlockSpec((_ROWS_BLK, D), lambda i: (i, 0)),
            pl.BlockSpec((_ROWS_BLK, D), lambda i: (i, 0)),
            pl.BlockSpec((_ROWS_BLK, D), lambda i: (i, 0)),
            pl.BlockSpec((_ROWS_BLK, 1), lambda i: (i, 0)),
            pl.BlockSpec((1, D), lambda i: (0, 0)),
        ],
        out_specs=pl.BlockSpec((_ROWS_BLK, D), lambda i: (i, 0)),
        out_shape=jax.ShapeDtypeStruct((N, D), jnp.float32),
    )(s0, s1, g, deg_col, b.reshape(1, D))

    return out
